# baseline (speedup 1.0000x reference)
import functools

import jax
import jax.numpy as jnp
from jax import lax
from jax.experimental import pallas as pl
from jax.experimental.pallas import tpu as pltpu
from jax.experimental.pallas import tpu_sc as plsc


def _ffn_body(xe_ref, w1_ref, b1_ref, w2_ref, b2_ref, out_ref, acc_ref):
    j = pl.program_id(1)
    nj = pl.num_programs(1)
    xb = xe_ref[0]
    w1b = w1_ref[0].astype(jnp.bfloat16)
    h = jax.lax.dot_general(
        xb, w1b, (((1,), (1,)), ((), ())),
        preferred_element_type=jnp.float32)
    h = h + b1_ref[0, 0, 0][None, :]
    h = 0.5 * h * (1.0 + jax.lax.erf(h * 0.7071067811865476))
    w2b = w2_ref[0].astype(jnp.bfloat16)
    part = jax.lax.dot_general(
        h.astype(jnp.bfloat16), w2b, (((1,), (1,)), ((), ())),
        preferred_element_type=jnp.float32)

    @pl.when(j == 0)
    def _():
        acc_ref[...] = part

    @pl.when(j != 0)
    def _():
        acc_ref[...] += part

    @pl.when(j == nj - 1)
    def _():
        out_ref[0] = (acc_ref[...] + b2_ref[0, 0][None, :]).astype(
            jnp.bfloat16)


def _ffn(xe, w1, b1r, w2, b2r, *, dt):
    e, c, d = xe.shape
    dff = w1.shape[1]
    nj = dff // dt
    return pl.pallas_call(
        _ffn_body,
        grid=(e, nj),
        in_specs=[
            pl.BlockSpec((1, c, d), lambda i, j: (i, 0, 0)),
            pl.BlockSpec((1, dt, d), lambda i, j: (i, j, 0)),
            pl.BlockSpec((1, 1, 1, dt), lambda i, j: (i, j, 0, 0)),
            pl.BlockSpec((1, d, dt), lambda i, j: (i, 0, j)),
            pl.BlockSpec((1, 1, d), lambda i, j: (i, 0, 0)),
        ],
        out_specs=pl.BlockSpec((1, c, d), lambda i, j: (i, 0, 0)),
        out_shape=jax.ShapeDtypeStruct((e, c, d), jnp.bfloat16),
        scratch_shapes=[pltpu.VMEM((c, d), jnp.float32)],
        compiler_params=pltpu.CompilerParams(
            dimension_semantics=("arbitrary", "arbitrary"),
        ),
    )(xe, w1, b1r, w2, b2r)


def _sc_gather(table, idx, *, chunk=64):
    v, w = table.shape
    bsz = idx.shape[0]
    info = plsc.get_sparse_core_info()
    nw = info.num_cores * info.num_subcores
    b_per_w = bsz // nw
    assert b_per_w * nw == bsz and b_per_w % chunk == 0
    nch = b_per_w // chunk
    mesh = plsc.VectorSubcoreMesh(core_axis_name="c", subcore_axis_name="s")

    @functools.partial(
        pl.kernel, mesh=mesh,
        out_type=jax.ShapeDtypeStruct((bsz, w), jnp.int32),
        scratch_types=[
            pltpu.VMEM((chunk,), jnp.int32),
            pltpu.VMEM((chunk, w), jnp.int32),
            pltpu.SemaphoreType.DMA,
        ],
    )
    def k(table_hbm, idx_hbm, out_hbm, idx_v, rows_v, sem):
        wid = lax.axis_index("s") * info.num_cores + lax.axis_index("c")
        base = wid * b_per_w

        def body(c, carry):
            off = base + c * chunk
            pltpu.sync_copy(idx_hbm.at[pl.ds(off, chunk)], idx_v)
            pltpu.async_copy(table_hbm.at[idx_v], rows_v, sem).wait()
            pltpu.sync_copy(rows_v, out_hbm.at[pl.ds(off, chunk)])
            return carry

        lax.fori_loop(0, nch, body, 0)

    return k(table, idx)


def _as_i32(a2d_bf16):
    n, d = a2d_bf16.shape
    return lax.bitcast_convert_type(
        a2d_bf16.reshape(n, d // 2, 2), jnp.int32)


def _as_bf16(a2d_i32):
    n, w = a2d_i32.shape
    return lax.bitcast_convert_type(a2d_i32, jnp.bfloat16).reshape(n, 2 * w)


def kernel(x, Wr, w1, b1, w2, b2):
    b, s, d = x.shape
    e = Wr.shape[0]
    x_flat = x.reshape(-1, d)
    n_tok = x_flat.shape[0]
    cap = int(1.25 * n_tok / e)

    router_logits = x_flat @ Wr.T
    router_probs = jax.nn.softmax(router_logits, axis=-1)
    gates = jnp.max(router_probs, axis=-1)
    indices = jnp.argmax(router_probs, axis=-1)

    expert_mask = jax.nn.one_hot(indices, e, dtype=jnp.float32)
    density = expert_mask.mean(axis=0)
    prob_mean = router_probs.mean(axis=0)
    load_balance_loss = e * jnp.sum(density * prob_mean) * 0.01
    router_z_loss = jnp.mean(
        jax.scipy.special.logsumexp(router_probs, axis=-1)) * 0.001
    aux_loss = load_balance_loss + router_z_loss

    tok = jnp.arange(n_tok, dtype=jnp.int32)
    idx32 = indices.astype(jnp.int32)
    sorted_e, _, sorted_tok = jax.lax.sort(
        (idx32, -gates, tok), num_keys=2, is_stable=True)
    counts = jnp.sum(expert_mask, axis=0).astype(jnp.int32)
    seg_start = jnp.concatenate(
        [jnp.zeros((1,), jnp.int32), jnp.cumsum(counts)[:-1].astype(jnp.int32)])
    pos = tok - seg_start[sorted_e]
    keep = pos < cap
    fslot = sorted_e * cap + pos
    slot_of_tok = jnp.full((n_tok,), -1, jnp.int32).at[sorted_tok].set(
        jnp.where(keep, fslot, -1))
    f_or_dummy = jnp.where(keep, fslot, e * cap)
    sel_idx = jnp.zeros((e * cap + 1,), jnp.int32).at[f_or_dummy].set(
        sorted_tok)[: e * cap]

    x_bf = x_flat.astype(jnp.bfloat16)
    xe = _as_bf16(_sc_gather(_as_i32(x_bf), sel_idx)).reshape(e, cap, d)
    eo = _ffn(xe, w1, b1.reshape(e, -1, 1, 1024), w2, b2.reshape(e, 1, d),
              dt=1024)
    eo_flat = eo.reshape(e * cap, d)
    comb_scale = jnp.where(slot_of_tok >= 0, gates, 0.0)
    src = jnp.maximum(slot_of_tok, 0)
    out_bf = _as_bf16(_sc_gather(_as_i32(eo_flat), src))
    out_flat = out_bf.astype(jnp.float32) * comb_scale[:, None]
    return out_flat.reshape(b, s, d), aux_loss


# speedup vs baseline: 2.5377x; 2.5377x over previous
import functools

import jax
import jax.numpy as jnp
from jax import lax
from jax.experimental import pallas as pl
from jax.experimental.pallas import tpu as pltpu
from jax.experimental.pallas import tpu_sc as plsc


def _ffn_body(xe_ref, w1_ref, b1_ref, w2_ref, b2_ref, g_ref, out_ref,
              acc_ref):
    j = pl.program_id(1)
    nj = pl.num_programs(1)
    xb = xe_ref[0].astype(jnp.bfloat16)
    w1b = w1_ref[0].astype(jnp.bfloat16)
    h = jax.lax.dot_general(
        xb, w1b, (((1,), (1,)), ((), ())),
        preferred_element_type=jnp.float32)
    h = h + b1_ref[0, 0, 0][None, :]
    h = 0.5 * h * (1.0 + jax.lax.erf(h * 0.7071067811865476))
    w2b = w2_ref[0].astype(jnp.bfloat16)
    part = jax.lax.dot_general(
        h.astype(jnp.bfloat16), w2b, (((1,), (1,)), ((), ())),
        preferred_element_type=jnp.float32)

    @pl.when(j == 0)
    def _():
        acc_ref[...] = part

    @pl.when(j != 0)
    def _():
        acc_ref[...] += part

    @pl.when(j == nj - 1)
    def _():
        out_ref[0] = (acc_ref[...] + b2_ref[0, 0][None, :]) * g_ref[0, 0][:, None]


def _ffn(xe, w1, b1r, w2, b2r, gr, *, dt):
    e, c, d = xe.shape
    dff = w1.shape[1]
    nj = dff // dt
    return pl.pallas_call(
        _ffn_body,
        grid=(e, nj),
        in_specs=[
            pl.BlockSpec((1, c, d), lambda i, j: (i, 0, 0)),
            pl.BlockSpec((1, dt, d), lambda i, j: (i, j, 0)),
            pl.BlockSpec((1, 1, 1, dt), lambda i, j: (i, j, 0, 0)),
            pl.BlockSpec((1, d, dt), lambda i, j: (i, 0, j)),
            pl.BlockSpec((1, 1, d), lambda i, j: (i, 0, 0)),
            pl.BlockSpec((1, 1, c), lambda i, j: (i, 0, 0)),
        ],
        out_specs=pl.BlockSpec((1, c, d), lambda i, j: (i, 0, 0)),
        out_shape=jax.ShapeDtypeStruct((e, c, d), jnp.float32),
        scratch_shapes=[pltpu.VMEM((c, d), jnp.float32)],
        compiler_params=pltpu.CompilerParams(
            dimension_semantics=("arbitrary", "arbitrary"),
        ),
    )(xe, w1, b1r, w2, b2r, gr)


def _sc_gather_rows(table, idx, *, chunk=32):
    v, w = table.shape
    bsz = idx.shape[0]
    info = plsc.get_sparse_core_info()
    nw = info.num_cores * info.num_subcores
    b_per_w = bsz // nw
    assert b_per_w * nw == bsz and b_per_w % chunk == 0
    nch = b_per_w // chunk
    mesh = plsc.VectorSubcoreMesh(core_axis_name="c", subcore_axis_name="s")

    out_type = [jax.ShapeDtypeStruct((bsz, w), jnp.float32)]
    scratch = [
        pltpu.VMEM((b_per_w,), jnp.int32),
        pltpu.VMEM((chunk, w), jnp.float32),
        pltpu.VMEM((chunk, w), jnp.float32),
        pltpu.SemaphoreType.DMA,
        pltpu.SemaphoreType.DMA,
        pltpu.SemaphoreType.DMA,
        pltpu.SemaphoreType.DMA,
    ]
    @functools.partial(pl.kernel, mesh=mesh, out_type=tuple(out_type),
                       scratch_types=scratch)
    def k(*refs):
        (table_hbm, idx_hbm, out_hbm, idx_v,
         r0, r1, sg0, sg1, sw0, sw1) = refs
        bufs = (r0, r1)
        gsems = (sg0, sg1)
        wsems = (sw0, sw1)
        wid = lax.axis_index("s") * info.num_cores + lax.axis_index("c")
        base = wid * b_per_w
        pltpu.sync_copy(idx_hbm.at[pl.ds(base, b_per_w)], idx_v)

        def start_gather(c):
            bi = c % 2
            return [pltpu.async_copy(
                table_hbm.at[idx_v.at[pl.ds(c * chunk, chunk)]],
                bufs[bi], gsems[bi])]

        def start_wb(c):
            bi = c % 2
            off = base + c * chunk
            return [pltpu.async_copy(
                bufs[bi], out_hbm.at[pl.ds(off, chunk)], wsems[bi])]

        pending_g = start_gather(0)
        pending_w = [None, None]
        for c in range(nch):
            bi = c % 2
            for cp in pending_g:
                cp.wait()
            if c + 1 < nch:
                if pending_w[1 - bi] is not None:
                    for cp in pending_w[1 - bi]:
                        cp.wait()
                    pending_w[1 - bi] = None
                pending_g = start_gather(c + 1)
            pending_w[bi] = start_wb(c)
        for pw in pending_w:
            if pw is not None:
                for cp in pw:
                    cp.wait()

    return k(table, idx)[0]


def kernel(x, Wr, w1, b1, w2, b2):
    b, s, d = x.shape
    e = Wr.shape[0]
    x_flat = x.reshape(-1, d)
    n_tok = x_flat.shape[0]
    cap = int(1.25 * n_tok / e)

    router_logits = x_flat @ Wr.T
    router_probs = jax.nn.softmax(router_logits, axis=-1)
    gates = jnp.max(router_probs, axis=-1)
    indices = jnp.argmax(router_probs, axis=-1)

    expert_mask = jax.nn.one_hot(indices, e, dtype=jnp.float32)
    density = expert_mask.mean(axis=0)
    prob_mean = router_probs.mean(axis=0)
    load_balance_loss = e * jnp.sum(density * prob_mean) * 0.01
    router_z_loss = jnp.mean(
        jax.scipy.special.logsumexp(router_probs, axis=-1)) * 0.001
    aux_loss = load_balance_loss + router_z_loss

    tok = jnp.arange(n_tok, dtype=jnp.int32)
    idx32 = indices.astype(jnp.int32)
    sorted_e, sorted_negg, sorted_tok = jax.lax.sort(
        (idx32, -gates, tok), num_keys=2, is_stable=True)
    counts = jnp.sum(expert_mask, axis=0).astype(jnp.int32)
    seg_start = jnp.concatenate(
        [jnp.zeros((1,), jnp.int32), jnp.cumsum(counts)[:-1].astype(jnp.int32)])
    pos = tok - seg_start[sorted_e]
    keep = pos < cap
    fslot = sorted_e * cap + pos
    slot_of_tok = jnp.full((n_tok,), -1, jnp.int32).at[sorted_tok].set(
        jnp.where(keep, fslot, -1))
    f_or_dummy = jnp.where(keep, fslot, e * cap)
    sel_idx = jnp.zeros((e * cap + 1,), jnp.int32).at[f_or_dummy].set(
        sorted_tok)[: e * cap]
    sel_gate = jnp.zeros((e * cap + 1,), jnp.float32).at[f_or_dummy].set(
        -sorted_negg)[: e * cap]

    xe_flat = _sc_gather_rows(x_flat, sel_idx)
    xe = xe_flat.reshape(e, cap, d)
    gr = sel_gate.reshape(e, 1, cap)

    eo = _ffn(xe, w1, b1.reshape(e, -1, 1, 1024), w2, b2.reshape(e, 1, d),
              gr, dt=1024)
    eo_flat = eo.reshape(e * cap, d)

    e_star = jnp.argmin(counts).astype(jnp.int32)
    pad_slot = e_star * cap + counts[e_star]
    src = jnp.where(slot_of_tok >= 0, slot_of_tok, pad_slot)
    out_flat = _sc_gather_rows(eo_flat, src)
    return out_flat.reshape(b, s, d), aux_loss


# speedup vs baseline: 2.6779x; 1.0552x over previous
import functools

import jax
import jax.numpy as jnp
from jax import lax
from jax.experimental import pallas as pl
from jax.experimental.pallas import tpu as pltpu
from jax.experimental.pallas import tpu_sc as plsc


def _ffn_body(xe_ref, w1_ref, b1_ref, w2_ref, b2_ref, g_ref, out_ref,
              acc_ref):
    j = pl.program_id(1)
    nj = pl.num_programs(1)
    xb = xe_ref[0].astype(jnp.bfloat16)
    w1b = w1_ref[0].astype(jnp.bfloat16)
    h = jax.lax.dot_general(
        xb, w1b, (((1,), (1,)), ((), ())),
        preferred_element_type=jnp.float32)
    h = h + b1_ref[0, 0, 0][None, :]
    h = 0.5 * h * (1.0 + jax.lax.erf(h * 0.7071067811865476))
    w2b = w2_ref[0].astype(jnp.bfloat16)
    part = jax.lax.dot_general(
        h.astype(jnp.bfloat16), w2b, (((1,), (1,)), ((), ())),
        preferred_element_type=jnp.float32)

    @pl.when(j == 0)
    def _():
        acc_ref[...] = part

    @pl.when(j != 0)
    def _():
        acc_ref[...] += part

    @pl.when(j == nj - 1)
    def _():
        out_ref[0] = (acc_ref[...] + b2_ref[0, 0][None, :]) * g_ref[0, 0][:, None]


def _ffn(xe, w1, b1r, w2, b2r, gr, *, dt):
    e, c, d = xe.shape
    dff = w1.shape[1]
    nj = dff // dt
    return pl.pallas_call(
        _ffn_body,
        grid=(e, nj),
        in_specs=[
            pl.BlockSpec((1, c, d), lambda i, j: (i, 0, 0)),
            pl.BlockSpec((1, dt, d), lambda i, j: (i, j, 0)),
            pl.BlockSpec((1, 1, 1, dt), lambda i, j: (i, j, 0, 0)),
            pl.BlockSpec((1, d, dt), lambda i, j: (i, 0, j)),
            pl.BlockSpec((1, 1, d), lambda i, j: (i, 0, 0)),
            pl.BlockSpec((1, 1, c), lambda i, j: (i, 0, 0)),
        ],
        out_specs=pl.BlockSpec((1, c, d), lambda i, j: (i, 0, 0)),
        out_shape=jax.ShapeDtypeStruct((e, c, d), jnp.float32),
        scratch_shapes=[pltpu.VMEM((c, d), jnp.float32)],
        compiler_params=pltpu.CompilerParams(
            dimension_semantics=("arbitrary", "arbitrary"),
        ),
    )(xe, w1, b1r, w2, b2r, gr)


def _sc_gather_rows(table, idx, *, chunk=32):
    v, w = table.shape
    bsz = idx.shape[0]
    info = plsc.get_sparse_core_info()
    nw = info.num_cores * info.num_subcores
    b_per_w = bsz // nw
    assert b_per_w * nw == bsz and b_per_w % chunk == 0
    nch = b_per_w // chunk
    mesh = plsc.VectorSubcoreMesh(core_axis_name="c", subcore_axis_name="s")

    out_type = [jax.ShapeDtypeStruct((bsz, w), jnp.float32)]
    scratch = [
        pltpu.VMEM((b_per_w,), jnp.int32),
        pltpu.VMEM((chunk, w), jnp.float32),
        pltpu.VMEM((chunk, w), jnp.float32),
        pltpu.SemaphoreType.DMA,
        pltpu.SemaphoreType.DMA,
        pltpu.SemaphoreType.DMA,
        pltpu.SemaphoreType.DMA,
    ]
    @functools.partial(pl.kernel, mesh=mesh, out_type=tuple(out_type),
                       scratch_types=scratch)
    def k(*refs):
        (table_hbm, idx_hbm, out_hbm, idx_v,
         r0, r1, sg0, sg1, sw0, sw1) = refs
        bufs = (r0, r1)
        gsems = (sg0, sg1)
        wsems = (sw0, sw1)
        wid = lax.axis_index("s") * info.num_cores + lax.axis_index("c")
        base = wid * b_per_w
        pltpu.sync_copy(idx_hbm.at[pl.ds(base, b_per_w)], idx_v)

        def start_gather(c):
            bi = c % 2
            return [pltpu.async_copy(
                table_hbm.at[idx_v.at[pl.ds(c * chunk, chunk)]],
                bufs[bi], gsems[bi])]

        def start_wb(c):
            bi = c % 2
            off = base + c * chunk
            return [pltpu.async_copy(
                bufs[bi], out_hbm.at[pl.ds(off, chunk)], wsems[bi])]

        pending_g = start_gather(0)
        pending_w = [None, None]
        for c in range(nch):
            bi = c % 2
            for cp in pending_g:
                cp.wait()
            if c + 1 < nch:
                if pending_w[1 - bi] is not None:
                    for cp in pending_w[1 - bi]:
                        cp.wait()
                    pending_w[1 - bi] = None
                pending_g = start_gather(c + 1)
            pending_w[bi] = start_wb(c)
        for pw in pending_w:
            if pw is not None:
                for cp in pw:
                    cp.wait()

    return k(table, idx)[0]


def _sc_return_scatter(eo_flat, src3, dst3, n_out, *, chunk=32):
    v, w = eo_flat.shape
    nw, nch, ck = src3.shape
    assert ck == chunk
    mesh = plsc.VectorSubcoreMesh(core_axis_name="c", subcore_axis_name="s")

    @functools.partial(
        pl.kernel, mesh=mesh,
        out_type=jax.ShapeDtypeStruct((n_out, w), jnp.float32),
        scratch_types=[
            pltpu.VMEM((nch, chunk), jnp.int32),
            pltpu.VMEM((nch, chunk), jnp.int32),
            pltpu.VMEM((chunk, w), jnp.float32),
            pltpu.VMEM((chunk, w), jnp.float32),
            pltpu.SemaphoreType.DMA,
            pltpu.SemaphoreType.DMA,
            pltpu.SemaphoreType.DMA,
            pltpu.SemaphoreType.DMA,
        ],
    )
    def k(eo_hbm, src_hbm, dst_hbm, out_hbm, src_v, dst_v,
          r0, r1, sg0, sg1, sw0, sw1):
        bufs = (r0, r1)
        gsems = (sg0, sg1)
        wsems = (sw0, sw1)
        wid = lax.axis_index("s") * 2 + lax.axis_index("c")
        pltpu.sync_copy(src_hbm.at[wid], src_v)
        pltpu.sync_copy(dst_hbm.at[wid], dst_v)

        def start_gather(c):
            bi = c % 2
            return pltpu.async_copy(
                eo_hbm.at[src_v.at[c]], bufs[bi], gsems[bi])

        def start_wb(c):
            bi = c % 2
            return pltpu.async_copy(
                bufs[bi], out_hbm.at[dst_v.at[c]], wsems[bi])

        pending_g = start_gather(0)
        pending_w = [None, None]
        for c in range(nch):
            bi = c % 2
            pending_g.wait()
            if c + 1 < nch:
                if pending_w[1 - bi] is not None:
                    pending_w[1 - bi].wait()
                    pending_w[1 - bi] = None
                pending_g = start_gather(c + 1)
            pending_w[bi] = start_wb(c)
        for pw in pending_w:
            if pw is not None:
                pw.wait()

    return k(eo_flat, src3, dst3)


def kernel(x, Wr, w1, b1, w2, b2):
    b, s, d = x.shape
    e = Wr.shape[0]
    x_flat = x.reshape(-1, d)
    n_tok = x_flat.shape[0]
    cap = int(1.25 * n_tok / e)

    router_logits = x_flat @ Wr.T
    router_probs = jax.nn.softmax(router_logits, axis=-1)
    gates = jnp.max(router_probs, axis=-1)
    indices = jnp.argmax(router_probs, axis=-1)

    expert_mask = jax.nn.one_hot(indices, e, dtype=jnp.float32)
    density = expert_mask.mean(axis=0)
    prob_mean = router_probs.mean(axis=0)
    load_balance_loss = e * jnp.sum(density * prob_mean) * 0.01
    router_z_loss = jnp.mean(
        jax.scipy.special.logsumexp(router_probs, axis=-1)) * 0.001
    aux_loss = load_balance_loss + router_z_loss

    tok = jnp.arange(n_tok, dtype=jnp.int32)
    idx32 = indices.astype(jnp.int32)
    sorted_e, sorted_negg, sorted_tok = jax.lax.sort(
        (idx32, -gates, tok), num_keys=2, is_stable=True)
    counts = jnp.sum(expert_mask, axis=0).astype(jnp.int32)
    seg_start = jnp.concatenate(
        [jnp.zeros((1,), jnp.int32), jnp.cumsum(counts)[:-1].astype(jnp.int32)])
    pos = tok - seg_start[sorted_e]
    keep = pos < cap
    fslot = sorted_e * cap + pos
    f_or_dummy = jnp.where(keep, fslot, e * cap)
    sel2 = jnp.zeros((e * cap + 1, 2), jnp.float32).at[f_or_dummy].set(
        jnp.stack([sorted_tok.astype(jnp.float32), -sorted_negg], axis=1)
    )[: e * cap]
    sel_idx = sel2[:, 0].astype(jnp.int32)
    sel_gate = sel2[:, 1]

    xe_flat = _sc_gather_rows(x_flat, sel_idx)
    xe = xe_flat.reshape(e, cap, d)
    gr = sel_gate.reshape(e, 1, cap)

    eo = _ffn(xe, w1, b1.reshape(e, -1, 1, 2048), w2, b2.reshape(e, 1, d),
              gr, dt=2048)
    eo_flat = eo.reshape(e * cap, d)

    e_star = jnp.argmin(counts).astype(jnp.int32)
    pad_slot = e_star * cap + counts[e_star]
    src_sorted = jnp.where(keep, fslot, pad_slot)
    nw = 32
    src3 = src_sorted.reshape(nw, -1, 32)
    dst3 = sorted_tok.reshape(nw, -1, 32)
    out_flat = _sc_return_scatter(eo_flat, src3, dst3, n_tok)
    return out_flat.reshape(b, s, d), aux_loss
